# baseline (speedup 1.0000x reference)
import numpy as np
import jax
import jax.numpy as jnp
from jax import lax
from jax.experimental import pallas as pl
from jax.experimental.pallas import tpu as pltpu
from jax.experimental.pallas import tpu_sc as plsc

_NV = 512
_NC = 64
_AP = 8
_B = 16384
_K = 128
_T = 4096
_G = _B // _T

_NCORE = 2
_NSUB = 16
_NW = _NCORE * _NSUB
_RPW = _B // _NW
_RBLK = 128
_NBLK = _RPW // _RBLK


def _build_tables():
    rng = np.random.default_rng(42)
    cons = []
    for c in range(_NC):
        pool = np.delete(np.arange(_NV), c)
        others = rng.choice(pool, size=_AP - 1, replace=False)
        body = [(int(c), float(rng.uniform(0.5, 1.5)), bool(rng.integers(0, 2)))]
        for v in others:
            body.append((int(v), float(rng.uniform(0.5, 1.5)),
                         bool(rng.integers(0, 2))))
        cons.append(body)
    masks = [b[1][0] for b in cons]
    atoms = [[(v, co * (-1.0 if s else 1.0)) for (v, co, s) in body[2:]]
             for body in cons]

    E = np.zeros((_NV, _K), np.float32)
    W = np.zeros((_NV, _K), np.float32)
    sidx = np.zeros((_NC,), np.int32)
    bidx = np.zeros((_NC,), np.int32)
    pidx = np.zeros((_NC,), np.int32)
    single_idx = {}
    k = 0
    for c in range(_NC):
        m = masks[c]
        if m < c:
            pk = single_idx[m]
            E[masks[m], k] = 1.0
            for v, w in atoms[m]:
                W[v, k] += w
            for v, w in atoms[c]:
                W[v, k] += w
            sidx[c] = k
            k += 1
            E[m, k] = 1.0
            for v, w in atoms[c]:
                W[v, k] += w
            bidx[c] = k
            pidx[c] = pk
            k += 1
        else:
            E[m, k] = 1.0
            for v, w in atoms[c]:
                W[v, k] += w
            sidx[c] = k
            bidx[c] = k
            pidx[c] = 0
            single_idx[c] = k
            k += 1
    return E, W, sidx, bidx, pidx


_E, _W, _SIDX, _BIDX, _PIDX = _build_tables()


def _tc_stage(p_ref, g_ref, e_ref, w_ref, cand_ref, ssc_ref, ssa_ref):
    i = pl.program_id(0)
    p = p_ref[:, :]
    cand = (jnp.dot(p, e_ref[:, :], preferred_element_type=jnp.float32)
            + jnp.dot(g_ref[:, :], w_ref[:, :],
                      preferred_element_type=jnp.float32))
    cand_ref[:, :] = cand
    ssc = jnp.sum(cand * cand, axis=0, keepdims=True)
    pa = p[:, :_K]
    ssa = jnp.sum(pa * pa, axis=0, keepdims=True)

    @pl.when(i == 0)
    def _():
        ssc_ref[:, :] = ssc
        ssa_ref[:, :] = ssa

    @pl.when(i != 0)
    def _():
        ssc_ref[:, :] = ssc_ref[:, :] + ssc
        ssa_ref[:, :] = ssa_ref[:, :] + ssa


def _sc_stage(p_hbm, cand_hbm, ssc_hbm, ssa_hbm, sidx_hbm, bidx_hbm,
              pidx_hbm, out_hbm, ssc_v, ssa_v, sidx_v, bidx_v, pidx_v,
              condf_v, p_blk, c_blk):
    wid = lax.axis_index("s") * _NCORE + lax.axis_index("c")
    base = wid * _RPW

    pltpu.sync_copy(ssc_hbm, ssc_v)
    pltpu.sync_copy(ssa_hbm, ssa_v)
    pltpu.sync_copy(sidx_hbm, sidx_v)
    pltpu.sync_copy(bidx_hbm, bidx_v)
    pltpu.sync_copy(pidx_hbm, pidx_v)

    for g in range(_NC // 16):
        sl = pl.ds(g * 16, 16)
        s_g = plsc.load_gather(ssc_v, [sidx_v[sl]])
        condf_v[sl] = jnp.where(s_g > ssa_v[sl], 1.0, 0.0)

    fired_gs = []
    ridx_gs = []
    for g in range(_NC // 16):
        sl = pl.ds(g * 16, 16)
        sidx_g = sidx_v[sl]
        bidx_g = bidx_v[sl]
        a_g = ssa_v[sl]
        cond_a = plsc.load_gather(ssc_v, [sidx_g]) > a_g
        cond_b = plsc.load_gather(ssc_v, [bidx_g]) > a_g
        cp = plsc.load_gather(condf_v, [pidx_v[sl]]) > 0.5
        fired_gs.append(jnp.where(cp, cond_a, cond_b))
        ridx_gs.append(jnp.where(cp, sidx_g, bidx_g))

    for blk in range(_NBLK):
        start = base + blk * _RBLK
        pltpu.sync_copy(p_hbm.at[pl.ds(start, _RBLK)], p_blk)
        pltpu.sync_copy(cand_hbm.at[pl.ds(start, _RBLK)], c_blk)

        def row_body(r, carry):
            rsplat = jnp.full((16,), r, jnp.int32)
            for g in range(_NC // 16):
                rv = plsc.load_gather(c_blk, [rsplat, ridx_gs[g]])
                pv = p_blk[r, pl.ds(g * 16, 16)]
                p_blk[r, pl.ds(g * 16, 16)] = jnp.where(fired_gs[g], rv, pv)
            return carry

        lax.fori_loop(0, _RBLK, row_body, 0)
        pltpu.sync_copy(p_blk, out_hbm.at[pl.ds(start, _RBLK)])


def kernel(preds, ground_truth):
    e = jnp.asarray(_E)
    w = jnp.asarray(_W)

    cand, ssc, ssa = pl.pallas_call(
        _tc_stage,
        grid=(_G,),
        in_specs=[
            pl.BlockSpec((_T, _NV), lambda i: (i, 0)),
            pl.BlockSpec((_T, _NV), lambda i: (i, 0)),
            pl.BlockSpec((_NV, _K), lambda i: (0, 0)),
            pl.BlockSpec((_NV, _K), lambda i: (0, 0)),
        ],
        out_specs=[
            pl.BlockSpec((_T, _K), lambda i: (i, 0)),
            pl.BlockSpec((1, _K), lambda i: (0, 0)),
            pl.BlockSpec((1, _K), lambda i: (0, 0)),
        ],
        out_shape=[
            jax.ShapeDtypeStruct((_B, _K), jnp.float32),
            jax.ShapeDtypeStruct((1, _K), jnp.float32),
            jax.ShapeDtypeStruct((1, _K), jnp.float32),
        ],
    )(preds, ground_truth, e, w)

    sc = pl.kernel(
        _sc_stage,
        out_type=jax.ShapeDtypeStruct((_B, _NV), jnp.float32),
        mesh=plsc.VectorSubcoreMesh(core_axis_name="c", subcore_axis_name="s"),
        compiler_params=pltpu.CompilerParams(needs_layout_passes=False),
        scratch_types=[
            pltpu.VMEM((_K,), jnp.float32),
            pltpu.VMEM((_K,), jnp.float32),
            pltpu.VMEM((_NC,), jnp.int32),
            pltpu.VMEM((_NC,), jnp.int32),
            pltpu.VMEM((_NC,), jnp.int32),
            pltpu.VMEM((_NC,), jnp.float32),
            pltpu.VMEM((_RBLK, _NV), jnp.float32),
            pltpu.VMEM((_RBLK, _K), jnp.float32),
        ],
    )
    return sc(preds, cand, ssc.reshape(_K), ssa.reshape(_K),
              jnp.asarray(_SIDX), jnp.asarray(_BIDX), jnp.asarray(_PIDX))


# speedup vs baseline: 1.2686x; 1.2686x over previous
import numpy as np
import jax
import jax.numpy as jnp
from jax import lax
from jax.experimental import pallas as pl
from jax.experimental.pallas import tpu as pltpu
from jax.experimental.pallas import tpu_sc as plsc

_NV = 512
_NC = 64
_AP = 8
_B = 16384
_K = 128
_T = 4096
_G = _B // _T

_NCORE = 2
_NSUB = 16


def _build_tables():
    rng = np.random.default_rng(42)
    cons = []
    for c in range(_NC):
        pool = np.delete(np.arange(_NV), c)
        others = rng.choice(pool, size=_AP - 1, replace=False)
        body = [(int(c), float(rng.uniform(0.5, 1.5)), bool(rng.integers(0, 2)))]
        for v in others:
            body.append((int(v), float(rng.uniform(0.5, 1.5)),
                         bool(rng.integers(0, 2))))
        cons.append(body)
    masks = [b[1][0] for b in cons]
    atoms = [[(v, co * (-1.0 if s else 1.0)) for (v, co, s) in body[2:]]
             for body in cons]

    E = np.zeros((_NV, _K), np.float32)
    W = np.zeros((_NV, _K), np.float32)
    M = np.zeros((_K, _K), np.float32)
    aidx = np.zeros((_K,), np.int32)
    typ = np.zeros((_K,), np.int32)
    cpidx = np.zeros((_K,), np.int32)
    sidx = np.zeros((_NC,), np.int32)
    bidx = np.zeros((_NC,), np.int32)
    pidx = np.zeros((_NC,), np.int32)
    single_idx = {}
    k = 0
    for c in range(_NC):
        m = masks[c]
        if m < c:
            pk = single_idx[m]
            E[masks[m], k] = 1.0
            for v, w in atoms[m]:
                W[v, k] += w
            for v, w in atoms[c]:
                W[v, k] += w
            M[k, c] = 1.0
            aidx[k] = c
            typ[k] = 1
            cpidx[k] = pk
            sidx[c] = k
            k += 1
            E[m, k] = 1.0
            for v, w in atoms[c]:
                W[v, k] += w
            M[k, c] = 1.0
            aidx[k] = c
            typ[k] = 2
            cpidx[k] = pk
            bidx[c] = k
            pidx[c] = pk
            k += 1
        else:
            E[m, k] = 1.0
            for v, w in atoms[c]:
                W[v, k] += w
            M[k, c] = 1.0
            aidx[k] = c
            typ[k] = 0
            sidx[c] = k
            bidx[c] = k
            pidx[c] = 0
            single_idx[c] = k
            k += 1
    return E, W, M, aidx, typ, cpidx, sidx, bidx, pidx


(_E, _W, _M, _AIDX, _TYP, _CPIDX, _SIDX, _BIDX, _PIDX) = _build_tables()


def _tc_stage1(p_ref, g_ref, e_ref, w_ref, out_ref, cand_ref, ssc_ref,
               ssa_ref):
    i = pl.program_id(0)
    p = p_ref[:, :]
    cand = (jnp.dot(p, e_ref[:, :], preferred_element_type=jnp.float32)
            + jnp.dot(g_ref[:, :], w_ref[:, :],
                      preferred_element_type=jnp.float32))
    out_ref[:, :] = p
    cand_ref[:, :] = cand
    ssc = jnp.sum(cand * cand, axis=0, keepdims=True)
    pa = p[:, :_K]
    ssa = jnp.sum(pa * pa, axis=0, keepdims=True)

    @pl.when(i == 0)
    def _():
        ssc_ref[:, :] = ssc
        ssa_ref[:, :] = ssa

    @pl.when(i != 0)
    def _():
        ssc_ref[:, :] = ssc_ref[:, :] + ssc
        ssa_ref[:, :] = ssa_ref[:, :] + ssa


def _sc_stage(ssc_hbm, ssa_hbm, aidx_hbm, typ_hbm, cpidx_hbm, sidx_hbm,
              bidx_hbm, pidx_hbm, sel_hbm, fired_hbm, ssc_v, ssa_v, aidx_v,
              typ_v, cpidx_v, sidx_v, bidx_v, pidx_v, rawf_v, sel_v, fired_v):
    wid = lax.axis_index("s") * _NCORE + lax.axis_index("c")

    @pl.when(wid == 0)
    def _():
        pltpu.sync_copy(ssc_hbm, ssc_v)
        pltpu.sync_copy(ssa_hbm, ssa_v)
        pltpu.sync_copy(aidx_hbm, aidx_v)
        pltpu.sync_copy(typ_hbm, typ_v)
        pltpu.sync_copy(cpidx_hbm, cpidx_v)
        pltpu.sync_copy(sidx_hbm, sidx_v)
        pltpu.sync_copy(bidx_hbm, bidx_v)
        pltpu.sync_copy(pidx_hbm, pidx_v)

        for g in range(_K // 16):
            sl = pl.ds(g * 16, 16)
            thr = plsc.load_gather(ssa_v, [aidx_v[sl]])
            rawf_v[sl] = jnp.where(ssc_v[sl] > thr, 1.0, 0.0)

        for g in range(_K // 16):
            sl = pl.ds(g * 16, 16)
            raw = rawf_v[sl]
            rawp = plsc.load_gather(rawf_v, [cpidx_v[sl]])
            t = typ_v[sl]
            gate = jnp.where(t == 1, rawp,
                             jnp.where(t == 2, 1.0 - rawp,
                                       jnp.full((16,), 1.0, jnp.float32)))
            sel_v[sl] = gate * raw

        for g in range(_NC // 16):
            sl = pl.ds(g * 16, 16)
            cp = plsc.load_gather(rawf_v, [pidx_v[sl]]) > 0.5
            ridx = jnp.where(cp, sidx_v[sl], bidx_v[sl])
            fired_v[sl] = plsc.load_gather(rawf_v, [ridx])
        for g in range(_NC // 16, _K // 16):
            fired_v[pl.ds(g * 16, 16)] = jnp.full((16,), 0.0, jnp.float32)

        pltpu.sync_copy(sel_v, sel_hbm)
        pltpu.sync_copy(fired_v, fired_hbm)


def _tc_stage2(p_ref, cand_ref, sel_ref, fired_ref, m_ref, prev_ref, out_ref):
    del prev_ref
    sel = sel_ref[:, :]
    fired = fired_ref[:, :]
    contrib = jnp.dot(cand_ref[:, :] * sel, m_ref[:, :],
                      preferred_element_type=jnp.float32)
    out_ref[:, :] = p_ref[:, :] * (1.0 - fired) + contrib


def kernel(preds, ground_truth):
    e = jnp.asarray(_E)
    w = jnp.asarray(_W)
    m = jnp.asarray(_M)

    out1, cand, ssc, ssa = pl.pallas_call(
        _tc_stage1,
        grid=(_G,),
        in_specs=[
            pl.BlockSpec((_T, _NV), lambda i: (i, 0)),
            pl.BlockSpec((_T, _NV), lambda i: (i, 0)),
            pl.BlockSpec((_NV, _K), lambda i: (0, 0)),
            pl.BlockSpec((_NV, _K), lambda i: (0, 0)),
        ],
        out_specs=[
            pl.BlockSpec((_T, _NV), lambda i: (i, 0)),
            pl.BlockSpec((_T, _K), lambda i: (i, 0)),
            pl.BlockSpec((1, _K), lambda i: (0, 0)),
            pl.BlockSpec((1, _K), lambda i: (0, 0)),
        ],
        out_shape=[
            jax.ShapeDtypeStruct((_B, _NV), jnp.float32),
            jax.ShapeDtypeStruct((_B, _K), jnp.float32),
            jax.ShapeDtypeStruct((1, _K), jnp.float32),
            jax.ShapeDtypeStruct((1, _K), jnp.float32),
        ],
    )(preds, ground_truth, e, w)

    sc = pl.kernel(
        _sc_stage,
        out_type=[
            jax.ShapeDtypeStruct((_K,), jnp.float32),
            jax.ShapeDtypeStruct((_K,), jnp.float32),
        ],
        mesh=plsc.VectorSubcoreMesh(core_axis_name="c", subcore_axis_name="s"),
        compiler_params=pltpu.CompilerParams(needs_layout_passes=False),
        scratch_types=[
            pltpu.VMEM((_K,), jnp.float32),
            pltpu.VMEM((_K,), jnp.float32),
            pltpu.VMEM((_K,), jnp.int32),
            pltpu.VMEM((_K,), jnp.int32),
            pltpu.VMEM((_K,), jnp.int32),
            pltpu.VMEM((_NC,), jnp.int32),
            pltpu.VMEM((_NC,), jnp.int32),
            pltpu.VMEM((_NC,), jnp.int32),
            pltpu.VMEM((_K,), jnp.float32),
            pltpu.VMEM((_K,), jnp.float32),
            pltpu.VMEM((_K,), jnp.float32),
        ],
    )
    sel, fired = sc(ssc.reshape(_K), ssa.reshape(_K), jnp.asarray(_AIDX),
                    jnp.asarray(_TYP), jnp.asarray(_CPIDX), jnp.asarray(_SIDX),
                    jnp.asarray(_BIDX), jnp.asarray(_PIDX))

    out = pl.pallas_call(
        _tc_stage2,
        grid=(_G,),
        in_specs=[
            pl.BlockSpec((_T, _K), lambda i: (i, 0)),
            pl.BlockSpec((_T, _K), lambda i: (i, 0)),
            pl.BlockSpec((1, _K), lambda i: (0, 0)),
            pl.BlockSpec((1, _K), lambda i: (0, 0)),
            pl.BlockSpec((_K, _K), lambda i: (0, 0)),
            pl.BlockSpec((8, _K), lambda i: (0, 0)),
        ],
        out_specs=pl.BlockSpec((_T, _K), lambda i: (i, 0)),
        out_shape=jax.ShapeDtypeStruct((_B, _NV), jnp.float32),
        input_output_aliases={5: 0},
    )(preds, cand, sel.reshape(1, _K), fired.reshape(1, _K), m, out1)
    return out


# speedup vs baseline: 1.2906x; 1.0174x over previous
import numpy as np
import jax
import jax.numpy as jnp
from jax import lax
from jax.experimental import pallas as pl
from jax.experimental.pallas import tpu as pltpu
from jax.experimental.pallas import tpu_sc as plsc

_NV = 512
_NC = 64
_AP = 8
_B = 16384
_K = 128
_T = 4096
_G = _B // _T

_NCORE = 2
_NSUB = 16


def _build_tables():
    rng = np.random.default_rng(42)
    cons = []
    for c in range(_NC):
        pool = np.delete(np.arange(_NV), c)
        others = rng.choice(pool, size=_AP - 1, replace=False)
        body = [(int(c), float(rng.uniform(0.5, 1.5)), bool(rng.integers(0, 2)))]
        for v in others:
            body.append((int(v), float(rng.uniform(0.5, 1.5)),
                         bool(rng.integers(0, 2))))
        cons.append(body)
    masks = [b[1][0] for b in cons]
    atoms = [[(v, co * (-1.0 if s else 1.0)) for (v, co, s) in body[2:]]
             for body in cons]

    E = np.zeros((_NV, _K), np.float32)
    W = np.zeros((_NV, _K), np.float32)
    M = np.zeros((_K, _K), np.float32)
    aidx = np.zeros((_K,), np.int32)
    typ = np.zeros((_K,), np.int32)
    cpidx = np.zeros((_K,), np.int32)
    sidx = np.zeros((_NC,), np.int32)
    bidx = np.zeros((_NC,), np.int32)
    pidx = np.zeros((_NC,), np.int32)
    single_idx = {}
    k = 0
    for c in range(_NC):
        m = masks[c]
        if m < c:
            pk = single_idx[m]
            E[masks[m], k] = 1.0
            for v, w in atoms[m]:
                W[v, k] += w
            for v, w in atoms[c]:
                W[v, k] += w
            M[k, c] = 1.0
            aidx[k] = c
            typ[k] = 1
            cpidx[k] = pk
            sidx[c] = k
            k += 1
            E[m, k] = 1.0
            for v, w in atoms[c]:
                W[v, k] += w
            M[k, c] = 1.0
            aidx[k] = c
            typ[k] = 2
            cpidx[k] = pk
            bidx[c] = k
            pidx[c] = pk
            k += 1
        else:
            E[m, k] = 1.0
            for v, w in atoms[c]:
                W[v, k] += w
            M[k, c] = 1.0
            aidx[k] = c
            typ[k] = 0
            sidx[c] = k
            bidx[c] = k
            pidx[c] = 0
            single_idx[c] = k
            k += 1
    return E, W, M, aidx, typ, cpidx, sidx, bidx, pidx


(_E, _W, _M, _AIDX, _TYP, _CPIDX, _SIDX, _BIDX, _PIDX) = _build_tables()


def _tc_stage1(p_ref, g_ref, e_ref, w_ref, out_ref, cand_ref, ssc_ref,
               ssa_ref):
    i = pl.program_id(0)
    p = p_ref[:, :]
    cand = (jnp.dot(p, e_ref[:, :], preferred_element_type=jnp.float32)
            + jnp.dot(g_ref[:, :], w_ref[:, :],
                      preferred_element_type=jnp.float32))
    out_ref[:, :] = p
    cand_ref[:, :] = cand
    ssc = jnp.sum(cand * cand, axis=0, keepdims=True)
    pa = p[:, :_K]
    ssa = jnp.sum(pa * pa, axis=0, keepdims=True)

    @pl.when(i == 0)
    def _():
        ssc_ref[:, :] = ssc
        ssa_ref[:, :] = ssa

    @pl.when(i != 0)
    def _():
        ssc_ref[:, :] = ssc_ref[:, :] + ssc
        ssa_ref[:, :] = ssa_ref[:, :] + ssa


def _sc_stage(ssc_hbm, ssa_hbm, aidx_hbm, typ_hbm, cpidx_hbm, sidx_hbm,
              bidx_hbm, pidx_hbm, sel_hbm, fired_hbm, ssc_v, ssa_v, aidx_v,
              typ_v, cpidx_v, sidx_v, bidx_v, pidx_v, rawf_v, sel_v, fired_v):
    wid = lax.axis_index("s") * _NCORE + lax.axis_index("c")

    @pl.when(wid == 0)
    def _():
        pltpu.sync_copy(ssc_hbm, ssc_v)
        pltpu.sync_copy(ssa_hbm, ssa_v)
        pltpu.sync_copy(aidx_hbm, aidx_v)
        pltpu.sync_copy(typ_hbm, typ_v)
        pltpu.sync_copy(cpidx_hbm, cpidx_v)
        pltpu.sync_copy(sidx_hbm, sidx_v)
        pltpu.sync_copy(bidx_hbm, bidx_v)
        pltpu.sync_copy(pidx_hbm, pidx_v)

        for g in range(_K // 16):
            sl = pl.ds(g * 16, 16)
            thr = plsc.load_gather(ssa_v, [aidx_v[sl]])
            rawf_v[sl] = jnp.where(ssc_v[sl] > thr, 1.0, 0.0)

        for g in range(_K // 16):
            sl = pl.ds(g * 16, 16)
            raw = rawf_v[sl]
            rawp = plsc.load_gather(rawf_v, [cpidx_v[sl]])
            t = typ_v[sl]
            gate = jnp.where(t == 1, rawp,
                             jnp.where(t == 2, 1.0 - rawp,
                                       jnp.full((16,), 1.0, jnp.float32)))
            sel_v[sl] = gate * raw

        for g in range(_NC // 16):
            sl = pl.ds(g * 16, 16)
            cp = plsc.load_gather(rawf_v, [pidx_v[sl]]) > 0.5
            ridx = jnp.where(cp, sidx_v[sl], bidx_v[sl])
            fired_v[sl] = plsc.load_gather(rawf_v, [ridx])
        for g in range(_NC // 16, _K // 16):
            fired_v[pl.ds(g * 16, 16)] = jnp.full((16,), 0.0, jnp.float32)

        pltpu.sync_copy(sel_v, sel_hbm)
        pltpu.sync_copy(fired_v, fired_hbm)


def _tc_stage2(p_ref, cand_ref, sel_ref, fired_ref, m_ref, prev_ref, out_ref):
    del prev_ref
    sel = sel_ref[:, :]
    fired = fired_ref[:, :]
    contrib = jnp.dot(cand_ref[:, :] * sel, m_ref[:, :],
                      preferred_element_type=jnp.float32)
    out_ref[:, :] = p_ref[:, :] * (1.0 - fired) + contrib


def kernel(preds, ground_truth):
    e = jnp.asarray(_E)
    w = jnp.asarray(_W)
    m = jnp.asarray(_M)

    out1, cand, ssc, ssa = pl.pallas_call(
        _tc_stage1,
        grid=(_G,),
        in_specs=[
            pl.BlockSpec((_T, _NV), lambda i: (i, 0)),
            pl.BlockSpec((_T, _NV), lambda i: (i, 0)),
            pl.BlockSpec((_NV, _K), lambda i: (0, 0)),
            pl.BlockSpec((_NV, _K), lambda i: (0, 0)),
        ],
        out_specs=[
            pl.BlockSpec((_T, _NV), lambda i: (i, 0)),
            pl.BlockSpec((_T, _K), lambda i: (i, 0)),
            pl.BlockSpec((1, _K), lambda i: (0, 0)),
            pl.BlockSpec((1, _K), lambda i: (0, 0)),
        ],
        out_shape=[
            jax.ShapeDtypeStruct((_B, _NV), jnp.float32),
            jax.ShapeDtypeStruct((_B, _K), jnp.float32),
            jax.ShapeDtypeStruct((1, _K), jnp.float32),
            jax.ShapeDtypeStruct((1, _K), jnp.float32),
        ],
    )(preds, ground_truth, e, w)

    sc = pl.kernel(
        _sc_stage,
        out_type=[
            jax.ShapeDtypeStruct((_K,), jnp.float32),
            jax.ShapeDtypeStruct((_K,), jnp.float32),
        ],
        mesh=plsc.VectorSubcoreMesh(core_axis_name="c", subcore_axis_name="s", num_cores=1),
        compiler_params=pltpu.CompilerParams(needs_layout_passes=False),
        scratch_types=[
            pltpu.VMEM((_K,), jnp.float32),
            pltpu.VMEM((_K,), jnp.float32),
            pltpu.VMEM((_K,), jnp.int32),
            pltpu.VMEM((_K,), jnp.int32),
            pltpu.VMEM((_K,), jnp.int32),
            pltpu.VMEM((_NC,), jnp.int32),
            pltpu.VMEM((_NC,), jnp.int32),
            pltpu.VMEM((_NC,), jnp.int32),
            pltpu.VMEM((_K,), jnp.float32),
            pltpu.VMEM((_K,), jnp.float32),
            pltpu.VMEM((_K,), jnp.float32),
        ],
    )
    sel, fired = sc(ssc.reshape(_K), ssa.reshape(_K), jnp.asarray(_AIDX),
                    jnp.asarray(_TYP), jnp.asarray(_CPIDX), jnp.asarray(_SIDX),
                    jnp.asarray(_BIDX), jnp.asarray(_PIDX))

    out = pl.pallas_call(
        _tc_stage2,
        grid=(_G,),
        in_specs=[
            pl.BlockSpec((_T, _K), lambda i: (i, 0)),
            pl.BlockSpec((_T, _K), lambda i: (i, 0)),
            pl.BlockSpec((1, _K), lambda i: (0, 0)),
            pl.BlockSpec((1, _K), lambda i: (0, 0)),
            pl.BlockSpec((_K, _K), lambda i: (0, 0)),
            pl.BlockSpec((8, _K), lambda i: (0, 0)),
        ],
        out_specs=pl.BlockSpec((_T, _K), lambda i: (i, 0)),
        out_shape=jax.ShapeDtypeStruct((_B, _NV), jnp.float32),
        input_output_aliases={5: 0},
    )(preds, cand, sel.reshape(1, _K), fired.reshape(1, _K), m, out1)
    return out


# speedup vs baseline: 1.3008x; 1.0079x over previous
import numpy as np
import jax
import jax.numpy as jnp
from jax import lax
from jax.experimental import pallas as pl
from jax.experimental.pallas import tpu as pltpu
from jax.experimental.pallas import tpu_sc as plsc

_NV = 512
_NC = 64
_AP = 8
_B = 16384
_K = 128
_T = 4096
_G = _B // _T

_NCORE = 2
_NSUB = 16


def _build_tables():
    rng = np.random.default_rng(42)
    cons = []
    for c in range(_NC):
        pool = np.delete(np.arange(_NV), c)
        others = rng.choice(pool, size=_AP - 1, replace=False)
        body = [(int(c), float(rng.uniform(0.5, 1.5)), bool(rng.integers(0, 2)))]
        for v in others:
            body.append((int(v), float(rng.uniform(0.5, 1.5)),
                         bool(rng.integers(0, 2))))
        cons.append(body)
    masks = [b[1][0] for b in cons]
    atoms = [[(v, co * (-1.0 if s else 1.0)) for (v, co, s) in body[2:]]
             for body in cons]

    E = np.zeros((_NV, _K), np.float32)
    W = np.zeros((_NV, _K), np.float32)
    M = np.zeros((_K, _K), np.float32)
    aidx = np.zeros((_K,), np.int32)
    typ = np.zeros((_K,), np.int32)
    cpidx = np.zeros((_K,), np.int32)
    sidx = np.zeros((_NC,), np.int32)
    bidx = np.zeros((_NC,), np.int32)
    pidx = np.zeros((_NC,), np.int32)
    single_idx = {}
    k = 0
    for c in range(_NC):
        m = masks[c]
        if m < c:
            pk = single_idx[m]
            E[masks[m], k] = 1.0
            for v, w in atoms[m]:
                W[v, k] += w
            for v, w in atoms[c]:
                W[v, k] += w
            M[k, c] = 1.0
            aidx[k] = c
            typ[k] = 1
            cpidx[k] = pk
            sidx[c] = k
            k += 1
            E[m, k] = 1.0
            for v, w in atoms[c]:
                W[v, k] += w
            M[k, c] = 1.0
            aidx[k] = c
            typ[k] = 2
            cpidx[k] = pk
            bidx[c] = k
            pidx[c] = pk
            k += 1
        else:
            E[m, k] = 1.0
            for v, w in atoms[c]:
                W[v, k] += w
            M[k, c] = 1.0
            aidx[k] = c
            typ[k] = 0
            sidx[c] = k
            bidx[c] = k
            pidx[c] = 0
            single_idx[c] = k
            k += 1
    return E, W, M, aidx, typ, cpidx, sidx, bidx, pidx


(_E, _W, _M, _AIDX, _TYP, _CPIDX, _SIDX, _BIDX, _PIDX) = _build_tables()


def _tc_stage1(p_ref, g_ref, e_ref, w_ref, out_ref, cand_ref, ssc_ref,
               ssa_ref):
    i = pl.program_id(0)
    p = p_ref[:, :]
    cand = (jnp.dot(p.astype(jnp.bfloat16), e_ref[:, :],
                    preferred_element_type=jnp.float32)
            + jnp.dot(g_ref[:, :].astype(jnp.bfloat16), w_ref[:, :],
                      preferred_element_type=jnp.float32))
    out_ref[:, :] = p
    cand_ref[:, :] = cand
    ssc = jnp.sum(cand * cand, axis=0, keepdims=True)
    pa = p[:, :_K]
    ssa = jnp.sum(pa * pa, axis=0, keepdims=True)

    @pl.when(i == 0)
    def _():
        ssc_ref[:, :] = ssc
        ssa_ref[:, :] = ssa

    @pl.when(i != 0)
    def _():
        ssc_ref[:, :] = ssc_ref[:, :] + ssc
        ssa_ref[:, :] = ssa_ref[:, :] + ssa


def _sc_stage(ssc_hbm, ssa_hbm, aidx_hbm, typ_hbm, cpidx_hbm, sidx_hbm,
              bidx_hbm, pidx_hbm, sel_hbm, fired_hbm, ssc_v, ssa_v, aidx_v,
              typ_v, cpidx_v, sidx_v, bidx_v, pidx_v, rawf_v, sel_v, fired_v):
    wid = lax.axis_index("s") * _NCORE + lax.axis_index("c")

    @pl.when(wid == 0)
    def _():
        pltpu.sync_copy(ssc_hbm, ssc_v)
        pltpu.sync_copy(ssa_hbm, ssa_v)
        pltpu.sync_copy(aidx_hbm, aidx_v)
        pltpu.sync_copy(typ_hbm, typ_v)
        pltpu.sync_copy(cpidx_hbm, cpidx_v)
        pltpu.sync_copy(sidx_hbm, sidx_v)
        pltpu.sync_copy(bidx_hbm, bidx_v)
        pltpu.sync_copy(pidx_hbm, pidx_v)

        for g in range(_K // 16):
            sl = pl.ds(g * 16, 16)
            thr = plsc.load_gather(ssa_v, [aidx_v[sl]])
            rawf_v[sl] = jnp.where(ssc_v[sl] > thr, 1.0, 0.0)

        for g in range(_K // 16):
            sl = pl.ds(g * 16, 16)
            raw = rawf_v[sl]
            rawp = plsc.load_gather(rawf_v, [cpidx_v[sl]])
            t = typ_v[sl]
            gate = jnp.where(t == 1, rawp,
                             jnp.where(t == 2, 1.0 - rawp,
                                       jnp.full((16,), 1.0, jnp.float32)))
            sel_v[sl] = gate * raw

        for g in range(_NC // 16):
            sl = pl.ds(g * 16, 16)
            cp = plsc.load_gather(rawf_v, [pidx_v[sl]]) > 0.5
            ridx = jnp.where(cp, sidx_v[sl], bidx_v[sl])
            fired_v[sl] = plsc.load_gather(rawf_v, [ridx])
        for g in range(_NC // 16, _K // 16):
            fired_v[pl.ds(g * 16, 16)] = jnp.full((16,), 0.0, jnp.float32)

        pltpu.sync_copy(sel_v, sel_hbm)
        pltpu.sync_copy(fired_v, fired_hbm)


def _tc_stage2(p_ref, cand_ref, sel_ref, fired_ref, m_ref, prev_ref, out_ref):
    del prev_ref
    sel = sel_ref[:, :]
    fired = fired_ref[:, :]
    contrib = jnp.dot(cand_ref[:, :] * sel, m_ref[:, :],
                      preferred_element_type=jnp.float32)
    out_ref[:, :] = p_ref[:, :] * (1.0 - fired) + contrib


def kernel(preds, ground_truth):
    e = jnp.asarray(_E, dtype=jnp.bfloat16)
    w = jnp.asarray(_W, dtype=jnp.bfloat16)
    m = jnp.asarray(_M)

    out1, cand, ssc, ssa = pl.pallas_call(
        _tc_stage1,
        grid=(_G,),
        in_specs=[
            pl.BlockSpec((_T, _NV), lambda i: (i, 0)),
            pl.BlockSpec((_T, _NV), lambda i: (i, 0)),
            pl.BlockSpec((_NV, _K), lambda i: (0, 0)),
            pl.BlockSpec((_NV, _K), lambda i: (0, 0)),
        ],
        out_specs=[
            pl.BlockSpec((_T, _NV), lambda i: (i, 0)),
            pl.BlockSpec((_T, _K), lambda i: (i, 0)),
            pl.BlockSpec((1, _K), lambda i: (0, 0)),
            pl.BlockSpec((1, _K), lambda i: (0, 0)),
        ],
        out_shape=[
            jax.ShapeDtypeStruct((_B, _NV), jnp.float32),
            jax.ShapeDtypeStruct((_B, _K), jnp.float32),
            jax.ShapeDtypeStruct((1, _K), jnp.float32),
            jax.ShapeDtypeStruct((1, _K), jnp.float32),
        ],
    )(preds, ground_truth, e, w)

    sc = pl.kernel(
        _sc_stage,
        out_type=[
            jax.ShapeDtypeStruct((_K,), jnp.float32),
            jax.ShapeDtypeStruct((_K,), jnp.float32),
        ],
        mesh=plsc.VectorSubcoreMesh(core_axis_name="c", subcore_axis_name="s", num_cores=1),
        compiler_params=pltpu.CompilerParams(needs_layout_passes=False),
        scratch_types=[
            pltpu.VMEM((_K,), jnp.float32),
            pltpu.VMEM((_K,), jnp.float32),
            pltpu.VMEM((_K,), jnp.int32),
            pltpu.VMEM((_K,), jnp.int32),
            pltpu.VMEM((_K,), jnp.int32),
            pltpu.VMEM((_NC,), jnp.int32),
            pltpu.VMEM((_NC,), jnp.int32),
            pltpu.VMEM((_NC,), jnp.int32),
            pltpu.VMEM((_K,), jnp.float32),
            pltpu.VMEM((_K,), jnp.float32),
            pltpu.VMEM((_K,), jnp.float32),
        ],
    )
    sel, fired = sc(ssc.reshape(_K), ssa.reshape(_K), jnp.asarray(_AIDX),
                    jnp.asarray(_TYP), jnp.asarray(_CPIDX), jnp.asarray(_SIDX),
                    jnp.asarray(_BIDX), jnp.asarray(_PIDX))

    out = pl.pallas_call(
        _tc_stage2,
        grid=(_G,),
        in_specs=[
            pl.BlockSpec((_T, _K), lambda i: (i, 0)),
            pl.BlockSpec((_T, _K), lambda i: (i, 0)),
            pl.BlockSpec((1, _K), lambda i: (0, 0)),
            pl.BlockSpec((1, _K), lambda i: (0, 0)),
            pl.BlockSpec((_K, _K), lambda i: (0, 0)),
            pl.BlockSpec((8, _K), lambda i: (0, 0)),
        ],
        out_specs=pl.BlockSpec((_T, _K), lambda i: (i, 0)),
        out_shape=jax.ShapeDtypeStruct((_B, _NV), jnp.float32),
        input_output_aliases={5: 0},
    )(preds, cand, sel.reshape(1, _K), fired.reshape(1, _K), m, out1)
    return out


# speedup vs baseline: 1.4065x; 1.0813x over previous
import numpy as np
import jax
import jax.numpy as jnp
from jax import lax
from jax.experimental import pallas as pl
from jax.experimental.pallas import tpu as pltpu
from jax.experimental.pallas import tpu_sc as plsc

_NV = 512
_NC = 64
_AP = 8
_B = 16384
_K = 128
_T = 4096
_G = _B // _T


def _build_tables():
    rng = np.random.default_rng(42)
    cons = []
    for c in range(_NC):
        pool = np.delete(np.arange(_NV), c)
        others = rng.choice(pool, size=_AP - 1, replace=False)
        body = [(int(c), float(rng.uniform(0.5, 1.5)), bool(rng.integers(0, 2)))]
        for v in others:
            body.append((int(v), float(rng.uniform(0.5, 1.5)),
                         bool(rng.integers(0, 2))))
        cons.append(body)
    masks = [b[1][0] for b in cons]
    atoms = [[(v, co * (-1.0 if s else 1.0)) for (v, co, s) in body[2:]]
             for body in cons]

    E = np.zeros((_NV, _K), np.float32)
    W = np.zeros((_NV, _K), np.float32)
    M = np.zeros((_K, _K), np.float32)
    aidx = np.zeros((_K,), np.int32)
    typ = np.zeros((_K,), np.int32)
    cpidx = np.zeros((_K,), np.int32)
    sidx = np.zeros((_K,), np.int32)
    bidx = np.zeros((_K,), np.int32)
    pidx = np.zeros((_K,), np.int32)
    single_idx = {}
    k = 0
    for c in range(_NC):
        m = masks[c]
        if m < c:
            pk = single_idx[m]
            E[masks[m], k] = 1.0
            for v, w in atoms[m]:
                W[v, k] += w
            for v, w in atoms[c]:
                W[v, k] += w
            M[k, c] = 1.0
            aidx[k] = _K + c
            typ[k] = 1
            cpidx[k] = pk
            sidx[c] = k
            k += 1
            E[m, k] = 1.0
            for v, w in atoms[c]:
                W[v, k] += w
            M[k, c] = 1.0
            aidx[k] = _K + c
            typ[k] = 2
            cpidx[k] = pk
            bidx[c] = k
            pidx[c] = pk
            k += 1
        else:
            E[m, k] = 1.0
            for v, w in atoms[c]:
                W[v, k] += w
            M[k, c] = 1.0
            aidx[k] = _K + c
            typ[k] = 0
            sidx[c] = k
            bidx[c] = k
            pidx[c] = 0
            single_idx[c] = k
            k += 1
    for kk in range(k, _K):
        aidx[kk] = _K
    tbl = np.concatenate([aidx, typ, cpidx, sidx, bidx, pidx])
    return E, W, M, tbl


_E, _W, _M, _TBL = _build_tables()


def _tc_stage1(p_ref, g_ref, e_ref, w_ref, out_ref, cand_ref, ss_ref):
    i = pl.program_id(0)
    p = p_ref[:, :]
    cand = (jnp.dot(p.astype(jnp.bfloat16), e_ref[:, :],
                    preferred_element_type=jnp.float32)
            + jnp.dot(g_ref[:, :].astype(jnp.bfloat16), w_ref[:, :],
                      preferred_element_type=jnp.float32))
    out_ref[:, :] = p
    cand_ref[:, :] = cand
    ssc = jnp.sum(cand * cand, axis=0, keepdims=True)
    pa = p[:, :_K]
    ssa = jnp.sum(pa * pa, axis=0, keepdims=True)
    ss = jnp.concatenate([ssc, ssa], axis=0)

    @pl.when(i == 0)
    def _():
        ss_ref[:, :] = ss

    @pl.when(i != 0)
    def _():
        ss_ref[:, :] = ss_ref[:, :] + ss


def _sc_stage(ss_hbm, tbl_hbm, out_hbm, ss_v, tbl_v, rawf_v, out_v):
    wid = lax.axis_index("s") * 2 + lax.axis_index("c")

    @pl.when(wid == 0)
    def _():
        pltpu.sync_copy(ss_hbm, ss_v)
        pltpu.sync_copy(tbl_hbm, tbl_v)

        def tslice(t, g):
            return tbl_v[pl.ds(t * _K + g * 16, 16)]

        for g in range(_K // 16):
            sl = pl.ds(g * 16, 16)
            thr = plsc.load_gather(ss_v, [tslice(0, g)])
            rawf_v[sl] = jnp.where(ss_v[sl] > thr, 1.0, 0.0)

        for g in range(_K // 16):
            sl = pl.ds(g * 16, 16)
            raw = rawf_v[sl]
            rawp = plsc.load_gather(rawf_v, [tslice(2, g)])
            t = tslice(1, g)
            gate = jnp.where(t == 1, rawp,
                             jnp.where(t == 2, 1.0 - rawp,
                                       jnp.full((16,), 1.0, jnp.float32)))
            out_v[sl] = gate * raw

        for g in range(_NC // 16):
            sl = pl.ds(_K + g * 16, 16)
            cp = plsc.load_gather(rawf_v, [tslice(5, g)]) > 0.5
            ridx = jnp.where(cp, tslice(3, g), tslice(4, g))
            out_v[sl] = plsc.load_gather(rawf_v, [ridx])
        for g in range(_NC // 16, _K // 16):
            out_v[pl.ds(_K + g * 16, 16)] = jnp.full((16,), 0.0, jnp.float32)

        pltpu.sync_copy(out_v, out_hbm)


def _tc_stage2(p_ref, cand_ref, sf_ref, m_ref, prev_ref, out_ref):
    del prev_ref
    sel = sf_ref[0:1, :]
    fired = sf_ref[1:2, :]
    contrib = jnp.dot(cand_ref[:, :] * sel, m_ref[:, :],
                      preferred_element_type=jnp.float32)
    out_ref[:, :] = p_ref[:, :] * (1.0 - fired) + contrib


def kernel(preds, ground_truth):
    e = jnp.asarray(_E, dtype=jnp.bfloat16)
    w = jnp.asarray(_W, dtype=jnp.bfloat16)
    m = jnp.asarray(_M)

    out1, cand, ss = pl.pallas_call(
        _tc_stage1,
        grid=(_G,),
        in_specs=[
            pl.BlockSpec((_T, _NV), lambda i: (i, 0)),
            pl.BlockSpec((_T, _NV), lambda i: (i, 0)),
            pl.BlockSpec((_NV, _K), lambda i: (0, 0)),
            pl.BlockSpec((_NV, _K), lambda i: (0, 0)),
        ],
        out_specs=[
            pl.BlockSpec((_T, _NV), lambda i: (i, 0)),
            pl.BlockSpec((_T, _K), lambda i: (i, 0)),
            pl.BlockSpec((2, _K), lambda i: (0, 0)),
        ],
        out_shape=[
            jax.ShapeDtypeStruct((_B, _NV), jnp.float32),
            jax.ShapeDtypeStruct((_B, _K), jnp.float32),
            jax.ShapeDtypeStruct((2, _K), jnp.float32),
        ],
    )(preds, ground_truth, e, w)

    sc = pl.kernel(
        _sc_stage,
        out_type=jax.ShapeDtypeStruct((2 * _K,), jnp.float32),
        mesh=plsc.VectorSubcoreMesh(core_axis_name="c", subcore_axis_name="s",
                                    num_cores=1),
        compiler_params=pltpu.CompilerParams(needs_layout_passes=False),
        scratch_types=[
            pltpu.VMEM((2 * _K,), jnp.float32),
            pltpu.VMEM((6 * _K,), jnp.int32),
            pltpu.VMEM((_K,), jnp.float32),
            pltpu.VMEM((2 * _K,), jnp.float32),
        ],
    )
    selfired = sc(ss.reshape(2 * _K), jnp.asarray(_TBL))

    out = pl.pallas_call(
        _tc_stage2,
        grid=(_G,),
        in_specs=[
            pl.BlockSpec((_T, _K), lambda i: (i, 0)),
            pl.BlockSpec((_T, _K), lambda i: (i, 0)),
            pl.BlockSpec((2, _K), lambda i: (0, 0)),
            pl.BlockSpec((_K, _K), lambda i: (0, 0)),
            pl.BlockSpec((8, _K), lambda i: (0, 0)),
        ],
        out_specs=pl.BlockSpec((_T, _K), lambda i: (i, 0)),
        out_shape=jax.ShapeDtypeStruct((_B, _NV), jnp.float32),
        input_output_aliases={4: 0},
    )(preds, cand, selfired.reshape(2, _K), m, out1)
    return out
